# Initial kernel scaffold; baseline (speedup 1.0000x reference)
import jax
import jax.numpy as jnp
from jax.experimental import pallas as pl


def kernel(ref_mel):
    raise NotImplementedError("write your pallas kernel here")


# speedup vs baseline: 1.4195x; 1.4195x over previous
import jax
import jax.numpy as jnp
from jax.experimental import pallas as pl
from jax.experimental.pallas import tpu as pltpu

B = 32
T = 4096
D = 80
BINS = 24
PADW = 4224
NCHUNK = 33
C_INV80 = 0.0125
C_INV5 = 0.2
C_INV7 = 0.142857149013519287109375


def _energy_kernel(x_ref, o_ref):
    o_ref[...] = jnp.sum(x_ref[...], axis=-1) * jnp.float32(C_INV80)


def _kth_bits(bits, k):
    def body(_, carry):
        lo, hi = carry
        mid = lo + ((hi - lo) >> 1)
        cnt = jnp.sum((bits <= mid).astype(jnp.int32), axis=1, keepdims=True)
        pred = cnt >= (k + 1)
        return (jnp.where(pred, lo, mid + 1), jnp.where(pred, mid, hi))
    lo0 = jnp.zeros((B, 1), jnp.int32)
    hi0 = jnp.full((B, 1), jnp.int32(0x7F7FFFFF))
    lo, _ = jax.lax.fori_loop(0, 31, body, (lo0, hi0))
    return lo


def _quantile_interp(vals, k):
    bits = jax.lax.bitcast_convert_type(vals, jnp.int32)
    a_bits = _kth_bits(bits, k)
    cnt_a = jnp.sum((bits <= a_bits).astype(jnp.int32), axis=1, keepdims=True)
    above = jnp.where(bits > a_bits, bits, jnp.int32(0x7FFFFFFF))
    b_bits = jnp.where(cnt_a >= (k + 2), a_bits, jnp.min(above, axis=1, keepdims=True))
    a = jax.lax.bitcast_convert_type(a_bits, jnp.float32)
    b = jax.lax.bitcast_convert_type(b_bits, jnp.float32)
    return a * jnp.float32(0.75) + b * jnp.float32(0.25)


def _blocked_cumsum(x2, scr):
    rows = x2.shape[0]
    r = x2.reshape(rows, NCHUNK, 128)
    acc = r[:, :, 0:1]
    scr[:, :, 0:1] = acc
    for l in range(1, 128):
        acc = acc + r[:, :, l:l + 1]
        scr[:, :, l:l + 1] = acc
    totals = acc[:, :, 0]
    cacc = jnp.zeros((rows, 1), jnp.float32)
    ccs = [cacc]
    for k in range(1, NCHUNK):
        cacc = cacc + totals[:, k - 1:k]
        ccs.append(cacc)
    carry = jnp.concatenate(ccs, axis=1)
    intra = scr[...]
    full = intra + carry[:, :, None]
    return full.reshape(rows, PADW)


def _main_kernel(e_ref, tp_ref, u_ref,
                 tp_out, tl_out, tb_out, ts_out, tv_out, st_out, scan_scr):
    e = e_ref[...]
    em = jnp.sum(e, axis=1, keepdims=True) * jnp.float32(2.0 ** -12)
    cen = e - em
    ss = jnp.sum(cen * cen, axis=1, keepdims=True)
    es = jnp.maximum(jnp.sqrt(ss * jnp.float32(1.0 / 4095.0)), 1e-6)
    ez = (e - em) / es

    delta = jnp.concatenate(
        [jnp.zeros((B, 1), jnp.float32), jnp.abs(e[:, 1:] - e[:, :-1])], axis=1)

    dthr = _quantile_interp(delta, 1433)
    pause = (ez <= jnp.float32(-0.5)) & (delta <= dthr)
    pf = pause.astype(jnp.float32)
    speech = jnp.float32(1.0) - pf
    voiced = (ez > jnp.float32(-0.1)).astype(jnp.float32)

    z = jnp.zeros((B, 128), jnp.float32)
    a5 = jnp.concatenate([z[:, :3], delta, z[:, :125]], axis=1)
    a7 = jnp.concatenate([z[:, :4], delta, z[:, :124]], axis=1)
    c = _blocked_cumsum(jnp.concatenate([a5, a7], axis=0), scan_scr)
    c5 = c[:B]
    c7 = c[B:]
    local_rate = (c5[:, 5:5 + T] - c5[:, 0:T]) * jnp.float32(C_INV5)
    bstr = (c7[:, 7:7 + T] - c7[:, 0:T]) * jnp.float32(C_INV7)

    bthr = _quantile_interp(bstr, 3071)
    bev = (bstr >= bthr).astype(jnp.float32)

    sp = speech
    sh = 1
    while sh < T:
        sp = sp + jnp.pad(sp, ((0, 0), (sh, 0)))[:, :T]
        sh *= 2
    total = jnp.maximum(sp[:, -1:], jnp.float32(1.0))
    progress = sp / total
    uniform = u_ref[0:1, :]
    sdb = progress - uniform

    def run_mean(mask_f):
        prev = jnp.pad(mask_f, ((0, 0), (1, 0)))[:, :T]
        starts = jnp.sum(((mask_f > 0.5) & (prev < 0.5)).astype(jnp.float32),
                         axis=1, keepdims=True)
        tot = jnp.sum(mask_f, axis=1, keepdims=True)
        return tot / jnp.maximum(starts, jnp.float32(1.0))

    pause_mean = jnp.sum(pf, axis=1, keepdims=True) * jnp.float32(2.0 ** -12)
    bev_mean = jnp.sum(bev, axis=1, keepdims=True) * jnp.float32(2.0 ** -12)
    voiced_mean = jnp.sum(voiced, axis=1, keepdims=True) * jnp.float32(2.0 ** -12)
    half = T // 2
    rate_trend = (jnp.sum(local_rate[:, half:], axis=1, keepdims=True) * jnp.float32(1.0 / half)
                  - jnp.sum(local_rate[:, :half], axis=1, keepdims=True) * jnp.float32(1.0 / half))
    stats = jnp.concatenate(
        [pause_mean, run_mean(pf), run_mean(speech), rate_trend,
         bev_mean, voiced_mean, jnp.zeros((B, 2), jnp.float32)], axis=1)
    st_out[...] = stats

    iota = jax.lax.broadcasted_iota(jnp.int32, (B, T), 1)
    feats = (pf, local_rate, bev, sdb, voiced)
    firsts = [f[:, 0:1] for f in feats]
    lasts = [f[:, -1:] for f in feats]
    outs = [[], [], [], [], []]
    for j in range(BINS):
        tpv = tp_ref[0, j]
        right = jnp.sum((progress < tpv).astype(jnp.int32), axis=1, keepdims=True)
        left = jnp.clip(right - 1, 0, T - 1)
        ri = jnp.clip(right, 0, T - 1)
        oh_l = (iota == left).astype(jnp.float32)
        oh_r = (iota == ri).astype(jnp.float32)
        lp = jnp.sum(progress * oh_l, axis=1, keepdims=True)
        rp = jnp.sum(progress * oh_r, axis=1, keepdims=True)
        denom = jnp.maximum(jnp.abs(rp - lp), jnp.float32(1e-6))
        alpha = jnp.clip((tpv - lp) / denom, 0.0, 1.0)
        lo_edge = right <= 0
        hi_edge = right >= T
        for fi in range(5):
            fl = jnp.sum(feats[fi] * oh_l, axis=1, keepdims=True)
            fr = jnp.sum(feats[fi] * oh_r, axis=1, keepdims=True)
            val = fl * (jnp.float32(1.0) - alpha) + fr * alpha
            val = jnp.where(lo_edge, firsts[fi], val)
            val = jnp.where(hi_edge, lasts[fi], val)
            outs[fi].append(val)
    tp_out[...] = jnp.concatenate(outs[0], axis=1)
    tl_out[...] = jnp.concatenate(outs[1], axis=1)
    tb_out[...] = jnp.concatenate(outs[2], axis=1)
    ts_out[...] = jnp.concatenate(outs[3], axis=1)
    tv_out[...] = jnp.concatenate(outs[4], axis=1)


def _run(ref_mel, interpret=False):
    x = ref_mel.astype(jnp.float32)
    e = pl.pallas_call(
        _energy_kernel,
        grid=(8,),
        in_specs=[pl.BlockSpec((B, 512, D), lambda i: (0, i, 0))],
        out_specs=pl.BlockSpec((B, 512), lambda i: (0, i)),
        out_shape=jax.ShapeDtypeStruct((B, T), jnp.float32),
        interpret=interpret,
    )(x)

    tp = jnp.linspace(0.0, 1.0, BINS)
    uni = jnp.linspace(0.0, 1.0, T)
    tp_pad = jnp.broadcast_to(jnp.pad(tp, (0, 128 - BINS))[None, :], (8, 128))
    u_pad = jnp.broadcast_to(uni[None, :], (8, T))

    outs = pl.pallas_call(
        _main_kernel,
        out_shape=(
            jax.ShapeDtypeStruct((B, BINS), jnp.float32),
            jax.ShapeDtypeStruct((B, BINS), jnp.float32),
            jax.ShapeDtypeStruct((B, BINS), jnp.float32),
            jax.ShapeDtypeStruct((B, BINS), jnp.float32),
            jax.ShapeDtypeStruct((B, BINS), jnp.float32),
            jax.ShapeDtypeStruct((B, 8), jnp.float32),
        ),
        scratch_shapes=[pltpu.VMEM((2 * B, NCHUNK, 128), jnp.float32)],
        interpret=interpret,
    )(e, tp_pad, u_pad)
    t_pf, t_lr, t_bev, t_sdb, t_voc, st = outs
    trace = jnp.stack([t_pf, t_lr, t_bev, t_sdb, t_voc], axis=-1)
    stats = st[:, :6]
    return trace, stats


def kernel(ref_mel):
    return _run(ref_mel)
